# baseline (speedup 1.0000x reference)
import functools

import jax
import jax.numpy as jnp
from jax import lax
from jax.experimental import pallas as pl
from jax.experimental.pallas import tpu as pltpu
from jax.experimental.pallas import tpu_sc as plsc

N_NODES = 10000
FEAT = 128
N_GRAPHS = 128
EDGES = 320000

NPAD = 10240
NBLK = NPAD // 128
NTILES = 32
CHUNK = 64
EPT = 10240
EPAD = NTILES * EPT
NCHUNK = EPT // CHUNK
NBUF = 4
SEG = 40
NSEG0 = 2
NSEG1 = 6
TOTCH = 16 * (NSEG0 + NSEG1) * SEG
RPT = NPAD // 16

_f32 = jnp.float32


@functools.lru_cache(maxsize=None)
def _sc_kernels():
    mesh = plsc.VectorSubcoreMesh(core_axis_name="c", subcore_axis_name="s")

    @functools.partial(
        pl.kernel,
        out_type=jax.ShapeDtypeStruct((2, NPAD), _f32),
        mesh=mesh,
        scratch_types=[
            pltpu.VMEM_SHARED((NPAD,), _f32),
            pltpu.VMEM((NCHUNK, CHUNK), jnp.int32),
            pltpu.VMEM((CHUNK,), _f32),
            pltpu.SemaphoreType.DMA,
        ],
    )
    def deg_kernel(dst_hbm, zero1_hbm, out_hbm, acc, didx, ones_v, dsem):
        cid = lax.axis_index("c")
        sid = lax.axis_index("s")
        wid = cid * 16 + sid
        for k in range(CHUNK // 16):
            ones_v[pl.ds(k * 16, 16)] = jnp.ones((16,), _f32)
        pltpu.sync_copy(dst_hbm.at[wid], didx)
        pltpu.sync_copy(zero1_hbm, acc.at[pl.ds(sid * RPT, RPT)])
        plsc.subcore_barrier()

        @pl.loop(0, NCHUNK, step=8)
        def _(c):
            for k in range(8):
                pltpu.async_copy(ones_v, acc.at[didx.at[c + k]], dsem, add=True)
            for k in range(8):
                pltpu.make_async_copy(ones_v, acc.at[didx.at[c + k]], dsem).wait()

        plsc.subcore_barrier()
        pltpu.sync_copy(acc.at[pl.ds(sid * RPT, RPT)],
                        out_hbm.at[cid, pl.ds(sid * RPT, RPT)])

    @functools.partial(
        pl.kernel,
        out_type=jax.ShapeDtypeStruct((2, NPAD, FEAT), _f32),
        mesh=mesh,
        scratch_types=[
            pltpu.VMEM_SHARED((NPAD, FEAT), _f32),
            pltpu.VMEM((SEG, 2, CHUNK), jnp.int32),
            [pltpu.VMEM((CHUNK, FEAT), _f32) for _ in range(NBUF)],
            [pltpu.SemaphoreType.DMA for _ in range(NBUF)],
            [pltpu.SemaphoreType.DMA for _ in range(NBUF)],
        ],
    )
    def spmm_kernel(h_hbm, sd_hbm, zero_hbm, out_hbm, acc, sd, rows, gsem, ssem):
        cid = lax.axis_index("c")
        sid = lax.axis_index("s")
        wid = cid * 16 + sid
        pltpu.sync_copy(zero_hbm.at[pl.ds(0, CHUNK)], rows[0])

        @pl.loop(0, RPT, step=CHUNK)
        def _(r):
            pltpu.sync_copy(rows[0], acc.at[pl.ds(sid * RPT + r, CHUNK)])

        plsc.subcore_barrier()

        def gfire(c, b):
            pltpu.async_copy(h_hbm.at[sd.at[c, 0]], rows[b], gsem[b])

        def gwait(c, b):
            pltpu.make_async_copy(h_hbm.at[sd.at[c, 0]], rows[b], gsem[b]).wait()

        def sfire(c, b):
            pltpu.async_copy(rows[b], acc.at[sd.at[c, 1]], ssem[b], add=True)

        def swait(c, b):
            pltpu.make_async_copy(rows[b], acc.at[sd.at[c, 1]], ssem[b]).wait()

        def edge_phase(chunk_base, nseg):
            for h in range(nseg):
                pltpu.sync_copy(sd_hbm.at[pl.ds(chunk_base + h * SEG, SEG)], sd)
                gfire(0, 0)
                gfire(1, 1)
                for c in (0, 1):
                    gfire(c + 2, c + 2)
                    gwait(c, c)
                    sfire(c, c)

                @pl.loop(0, (SEG - 4) // NBUF)
                def _(g):
                    for b in range(NBUF):
                        c = g * NBUF + b + 2
                        bb = (2 + b) % NBUF
                        swait(c - 2, b)
                        gfire(c + 2, b)
                        gwait(c, bb)
                        sfire(c, bb)

                for c in (SEG - 2, SEG - 1):
                    gwait(c, c % NBUF)
                    sfire(c, c % NBUF)
                for b in range(NBUF):
                    swait(b, b)

        @pl.when(cid == 0)
        def _():
            edge_phase(sid * (NSEG0 * SEG), NSEG0)

        @pl.when(cid == 1)
        def _():
            edge_phase(16 * NSEG0 * SEG + sid * (NSEG1 * SEG), NSEG1)

        plsc.subcore_barrier()

        @pl.loop(0, RPT, step=2 * CHUNK)
        def _(r):
            for b in range(2):
                rr = sid * RPT + r + b * CHUNK
                pltpu.sync_copy(acc.at[pl.ds(rr, CHUNK)], rows[b])
                pltpu.async_copy(rows[b], out_hbm.at[cid, pl.ds(rr, CHUNK)],
                                 gsem[b])
            for b in range(2):
                rr = sid * RPT + r + b * CHUNK
                pltpu.make_async_copy(rows[b], out_hbm.at[cid, pl.ds(rr, CHUNK)],
                                      gsem[b]).wait()

    return deg_kernel, spmm_kernel


def _prep_body(degp_ref, x_ref, w_ref, h_ref, dis_ref):
    deg = degp_ref[0] + degp_ref[1] + 1.0
    dis = lax.rsqrt(deg)
    h = jnp.dot(x_ref[...], w_ref[...], preferred_element_type=_f32)
    h_ref[...] = h * dis
    dis_ref[...] = dis


def _combine_body(p_ref, h_ref, dis_ref, b_ref, w_ref, out_ref):
    a = dis_ref[...] * (p_ref[0] + p_ref[1] + h_ref[...]) + b_ref[...]
    a = jnp.maximum(a, 0.0)
    out_ref[...] = jnp.dot(a, w_ref[...], preferred_element_type=_f32) * dis_ref[...]


def _final_body(p_ref, h_ref, dis_ref, b_ref, batch_ref, linw_ref, linb_ref,
                out_ref, sums, cnt):
    i = pl.program_id(0)

    @pl.when(i == 0)
    def _():
        sums[...] = jnp.zeros_like(sums)
        cnt[...] = jnp.zeros_like(cnt)

    a = dis_ref[...] * (p_ref[0] + p_ref[1] + h_ref[...]) + b_ref[...]
    a = jnp.maximum(a, 0.0)
    onehot = (batch_ref[...] == lax.broadcasted_iota(jnp.int32, (1, N_GRAPHS), 1))
    onehot = onehot.astype(_f32)
    dn = (((0,), (0,)), ((), ()))
    hi = lax.Precision.HIGHEST
    sums[...] += lax.dot_general(onehot, a, dn, precision=hi,
                                 preferred_element_type=_f32)
    cnt[...] += lax.dot_general(onehot, jnp.ones((128, 1), _f32), dn,
                                precision=hi, preferred_element_type=_f32)

    @pl.when(i == pl.num_programs(0) - 1)
    def _():
        pooled = sums[...] / jnp.maximum(cnt[...], 1.0)
        out_ref[...] = jnp.dot(pooled, linw_ref[...],
                               preferred_element_type=_f32) + linb_ref[...]


def _prep_call(degp3, x_p, W1):
    return pl.pallas_call(
        _prep_body,
        grid=(NBLK,),
        in_specs=[
            pl.BlockSpec((2, 128, 1), lambda i: (0, i, 0)),
            pl.BlockSpec((128, FEAT), lambda i: (i, 0)),
            pl.BlockSpec((FEAT, FEAT), lambda i: (0, 0)),
        ],
        out_specs=[
            pl.BlockSpec((128, FEAT), lambda i: (i, 0)),
            pl.BlockSpec((128, 1), lambda i: (i, 0)),
        ],
        out_shape=[
            jax.ShapeDtypeStruct((NPAD, FEAT), _f32),
            jax.ShapeDtypeStruct((NPAD, 1), _f32),
        ],
    )(degp3, x_p, W1)


def _combine_call(p, h, dis, b_row, W_next):
    return pl.pallas_call(
        _combine_body,
        grid=(NBLK,),
        in_specs=[
            pl.BlockSpec((2, 128, FEAT), lambda i: (0, i, 0)),
            pl.BlockSpec((128, FEAT), lambda i: (i, 0)),
            pl.BlockSpec((128, 1), lambda i: (i, 0)),
            pl.BlockSpec((1, FEAT), lambda i: (0, 0)),
            pl.BlockSpec((FEAT, FEAT), lambda i: (0, 0)),
        ],
        out_specs=pl.BlockSpec((128, FEAT), lambda i: (i, 0)),
        out_shape=jax.ShapeDtypeStruct((NPAD, FEAT), _f32),
    )(p, h, dis, b_row, W_next)


def _final_call(p, h, dis, b_row, batch_p, linw_p, linb_p):
    return pl.pallas_call(
        _final_body,
        grid=(NBLK,),
        in_specs=[
            pl.BlockSpec((2, 128, FEAT), lambda i: (0, i, 0)),
            pl.BlockSpec((128, FEAT), lambda i: (i, 0)),
            pl.BlockSpec((128, 1), lambda i: (i, 0)),
            pl.BlockSpec((1, FEAT), lambda i: (0, 0)),
            pl.BlockSpec((128, 1), lambda i: (i, 0)),
            pl.BlockSpec((FEAT, FEAT), lambda i: (0, 0)),
            pl.BlockSpec((1, FEAT), lambda i: (0, 0)),
        ],
        out_specs=pl.BlockSpec((N_GRAPHS, FEAT), lambda i: (0, 0)),
        out_shape=jax.ShapeDtypeStruct((N_GRAPHS, FEAT), _f32),
        scratch_shapes=[
            pltpu.VMEM((N_GRAPHS, FEAT), _f32),
            pltpu.VMEM((N_GRAPHS, 1), _f32),
        ],
    )(p, h, dis, b_row, batch_p, linw_p, linb_p)


def kernel(x, edge_index, batch, W1, b1, W2, b2, W3, b3, lin_W, lin_b):
    deg_call, spmm_call = _sc_kernels()

    sink = jnp.full((EPAD - EDGES,), N_NODES, jnp.int32)
    src_f = jnp.concatenate([edge_index[0], sink])
    dst_f = jnp.concatenate([edge_index[1], sink])
    dst_p = dst_f.reshape(NTILES, NCHUNK, CHUNK)
    sd_p = jnp.stack([src_f.reshape(TOTCH, CHUNK),
                      dst_f.reshape(TOTCH, CHUNK)], axis=1)
    x_p = jnp.pad(x, ((0, NPAD - N_NODES), (0, 0)))
    zero1 = jnp.zeros((RPT,), _f32)
    zero2 = jnp.zeros((RPT, FEAT), _f32)
    batch_p = jnp.concatenate(
        [batch, jnp.full((NPAD - N_NODES,), N_GRAPHS, batch.dtype)]
    ).reshape(NPAD, 1)
    linw_p = jnp.pad(lin_W, ((0, 0), (0, FEAT - lin_W.shape[1])))
    linb_p = jnp.broadcast_to(lin_b.reshape(1, 1), (1, FEAT))

    degp3 = deg_call(dst_p, zero1).reshape(2, NPAD, 1)
    h1, dis = _prep_call(degp3, x_p, W1)
    p1 = spmm_call(h1, sd_p, zero2)
    h2 = _combine_call(p1, h1, dis, b1.reshape(1, FEAT), W2)
    p2 = spmm_call(h2, sd_p, zero2)
    h3 = _combine_call(p2, h2, dis, b2.reshape(1, FEAT), W3)
    p3 = spmm_call(h3, sd_p, zero2)
    outm = _final_call(p3, h3, dis, b3.reshape(1, FEAT), batch_p, linw_p, linb_p)
    return outm[:, :1]


# speedup vs baseline: 1.0800x; 1.0800x over previous
import functools

import jax
import jax.numpy as jnp
from jax import lax
from jax.experimental import pallas as pl
from jax.experimental.pallas import tpu as pltpu
from jax.experimental.pallas import tpu_sc as plsc

N_NODES = 10000
FEAT = 128
N_GRAPHS = 128
EDGES = 320000

NPAD = 10240
NBLK = NPAD // 128
NTILES = 32
CHUNK = 64
EPT = 10240
EPAD = NTILES * EPT
NCHUNK = EPT // CHUNK
NBUF = 4
SEG = 40
NSEG0 = 6
NSEG1 = 2
TOTCH = 16 * (NSEG0 + NSEG1) * SEG
RPT = NPAD // 16

_f32 = jnp.float32


@functools.lru_cache(maxsize=None)
def _sc_kernels():
    mesh = plsc.VectorSubcoreMesh(core_axis_name="c", subcore_axis_name="s")

    @functools.partial(
        pl.kernel,
        out_type=jax.ShapeDtypeStruct((2, NPAD), _f32),
        mesh=mesh,
        scratch_types=[
            pltpu.VMEM_SHARED((NPAD,), _f32),
            pltpu.VMEM((NCHUNK, CHUNK), jnp.int32),
            pltpu.VMEM((CHUNK,), _f32),
            pltpu.SemaphoreType.DMA,
        ],
    )
    def deg_kernel(dst_hbm, zero1_hbm, out_hbm, acc, didx, ones_v, dsem):
        cid = lax.axis_index("c")
        sid = lax.axis_index("s")
        wid = cid * 16 + sid
        for k in range(CHUNK // 16):
            ones_v[pl.ds(k * 16, 16)] = jnp.ones((16,), _f32)
        pltpu.sync_copy(dst_hbm.at[wid], didx)
        pltpu.sync_copy(zero1_hbm, acc.at[pl.ds(sid * RPT, RPT)])
        plsc.subcore_barrier()

        @pl.loop(0, NCHUNK, step=8)
        def _(c):
            for k in range(8):
                pltpu.async_copy(ones_v, acc.at[didx.at[c + k]], dsem, add=True)
            for k in range(8):
                pltpu.make_async_copy(ones_v, acc.at[didx.at[c + k]], dsem).wait()

        plsc.subcore_barrier()
        pltpu.sync_copy(acc.at[pl.ds(sid * RPT, RPT)],
                        out_hbm.at[cid, pl.ds(sid * RPT, RPT)])

    @functools.partial(
        pl.kernel,
        out_type=jax.ShapeDtypeStruct((2, NPAD, FEAT), _f32),
        mesh=mesh,
        scratch_types=[
            pltpu.VMEM_SHARED((NPAD, FEAT), _f32),
            pltpu.VMEM((SEG, 2, CHUNK), jnp.int32),
            [pltpu.VMEM((CHUNK, FEAT), _f32) for _ in range(NBUF)],
            [pltpu.SemaphoreType.DMA for _ in range(NBUF)],
            [pltpu.SemaphoreType.DMA for _ in range(NBUF)],
        ],
    )
    def spmm_kernel(h_hbm, sd_hbm, zero_hbm, out_hbm, acc, sd, rows, gsem, ssem):
        cid = lax.axis_index("c")
        sid = lax.axis_index("s")
        wid = cid * 16 + sid
        pltpu.sync_copy(zero_hbm.at[pl.ds(0, CHUNK)], rows[0])

        @pl.loop(0, RPT, step=CHUNK)
        def _(r):
            pltpu.sync_copy(rows[0], acc.at[pl.ds(sid * RPT + r, CHUNK)])

        plsc.subcore_barrier()

        def gfire(c, b):
            pltpu.async_copy(h_hbm.at[sd.at[c, 0]], rows[b], gsem[b])

        def gwait(c, b):
            pltpu.make_async_copy(h_hbm.at[sd.at[c, 0]], rows[b], gsem[b]).wait()

        def sfire(c, b):
            pltpu.async_copy(rows[b], acc.at[sd.at[c, 1]], ssem[b], add=True)

        def swait(c, b):
            pltpu.make_async_copy(rows[b], acc.at[sd.at[c, 1]], ssem[b]).wait()

        def edge_phase(chunk_base, nseg):
            for h in range(nseg):
                pltpu.sync_copy(sd_hbm.at[pl.ds(chunk_base + h * SEG, SEG)], sd)
                gfire(0, 0)
                gfire(1, 1)
                for c in (0, 1):
                    gfire(c + 2, c + 2)
                    gwait(c, c)
                    sfire(c, c)

                @pl.loop(0, (SEG - 4) // NBUF)
                def _(g):
                    for b in range(NBUF):
                        c = g * NBUF + b + 2
                        bb = (2 + b) % NBUF
                        swait(c - 2, b)
                        gfire(c + 2, b)
                        gwait(c, bb)
                        sfire(c, bb)

                for c in (SEG - 2, SEG - 1):
                    gwait(c, c % NBUF)
                    sfire(c, c % NBUF)
                for b in range(NBUF):
                    swait(b, b)

        @pl.when(cid == 0)
        def _():
            edge_phase(sid * (NSEG0 * SEG), NSEG0)

        @pl.when(cid == 1)
        def _():
            edge_phase(16 * NSEG0 * SEG + sid * (NSEG1 * SEG), NSEG1)

        plsc.subcore_barrier()

        @pl.loop(0, RPT, step=2 * CHUNK)
        def _(r):
            for b in range(2):
                rr = sid * RPT + r + b * CHUNK
                pltpu.sync_copy(acc.at[pl.ds(rr, CHUNK)], rows[b])
                pltpu.async_copy(rows[b], out_hbm.at[cid, pl.ds(rr, CHUNK)],
                                 gsem[b])
            for b in range(2):
                rr = sid * RPT + r + b * CHUNK
                pltpu.make_async_copy(rows[b], out_hbm.at[cid, pl.ds(rr, CHUNK)],
                                      gsem[b]).wait()

    return deg_kernel, spmm_kernel


def _prep_body(degp_ref, x_ref, w_ref, h_ref, dis_ref):
    deg = degp_ref[0] + degp_ref[1] + 1.0
    dis = lax.rsqrt(deg)
    h = jnp.dot(x_ref[...], w_ref[...], preferred_element_type=_f32)
    h_ref[...] = h * dis
    dis_ref[...] = dis


def _combine_body(p_ref, h_ref, dis_ref, b_ref, w_ref, out_ref):
    a = dis_ref[...] * (p_ref[0] + p_ref[1] + h_ref[...]) + b_ref[...]
    a = jnp.maximum(a, 0.0)
    out_ref[...] = jnp.dot(a, w_ref[...], preferred_element_type=_f32) * dis_ref[...]


def _final_body(p_ref, h_ref, dis_ref, b_ref, batch_ref, linw_ref, linb_ref,
                out_ref, sums, cnt):
    i = pl.program_id(0)

    @pl.when(i == 0)
    def _():
        sums[...] = jnp.zeros_like(sums)
        cnt[...] = jnp.zeros_like(cnt)

    a = dis_ref[...] * (p_ref[0] + p_ref[1] + h_ref[...]) + b_ref[...]
    a = jnp.maximum(a, 0.0)
    onehot = (batch_ref[...] == lax.broadcasted_iota(jnp.int32, (1, N_GRAPHS), 1))
    onehot = onehot.astype(_f32)
    dn = (((0,), (0,)), ((), ()))
    hi = lax.Precision.HIGHEST
    sums[...] += lax.dot_general(onehot, a, dn, precision=hi,
                                 preferred_element_type=_f32)
    cnt[...] += lax.dot_general(onehot, jnp.ones((128, 1), _f32), dn,
                                precision=hi, preferred_element_type=_f32)

    @pl.when(i == pl.num_programs(0) - 1)
    def _():
        pooled = sums[...] / jnp.maximum(cnt[...], 1.0)
        out_ref[...] = jnp.dot(pooled, linw_ref[...],
                               preferred_element_type=_f32) + linb_ref[...]


def _prep_call(degp3, x_p, W1):
    return pl.pallas_call(
        _prep_body,
        grid=(NBLK,),
        in_specs=[
            pl.BlockSpec((2, 128, 1), lambda i: (0, i, 0)),
            pl.BlockSpec((128, FEAT), lambda i: (i, 0)),
            pl.BlockSpec((FEAT, FEAT), lambda i: (0, 0)),
        ],
        out_specs=[
            pl.BlockSpec((128, FEAT), lambda i: (i, 0)),
            pl.BlockSpec((128, 1), lambda i: (i, 0)),
        ],
        out_shape=[
            jax.ShapeDtypeStruct((NPAD, FEAT), _f32),
            jax.ShapeDtypeStruct((NPAD, 1), _f32),
        ],
    )(degp3, x_p, W1)


def _combine_call(p, h, dis, b_row, W_next):
    return pl.pallas_call(
        _combine_body,
        grid=(NBLK,),
        in_specs=[
            pl.BlockSpec((2, 128, FEAT), lambda i: (0, i, 0)),
            pl.BlockSpec((128, FEAT), lambda i: (i, 0)),
            pl.BlockSpec((128, 1), lambda i: (i, 0)),
            pl.BlockSpec((1, FEAT), lambda i: (0, 0)),
            pl.BlockSpec((FEAT, FEAT), lambda i: (0, 0)),
        ],
        out_specs=pl.BlockSpec((128, FEAT), lambda i: (i, 0)),
        out_shape=jax.ShapeDtypeStruct((NPAD, FEAT), _f32),
    )(p, h, dis, b_row, W_next)


def _final_call(p, h, dis, b_row, batch_p, linw_p, linb_p):
    return pl.pallas_call(
        _final_body,
        grid=(NBLK,),
        in_specs=[
            pl.BlockSpec((2, 128, FEAT), lambda i: (0, i, 0)),
            pl.BlockSpec((128, FEAT), lambda i: (i, 0)),
            pl.BlockSpec((128, 1), lambda i: (i, 0)),
            pl.BlockSpec((1, FEAT), lambda i: (0, 0)),
            pl.BlockSpec((128, 1), lambda i: (i, 0)),
            pl.BlockSpec((FEAT, FEAT), lambda i: (0, 0)),
            pl.BlockSpec((1, FEAT), lambda i: (0, 0)),
        ],
        out_specs=pl.BlockSpec((N_GRAPHS, FEAT), lambda i: (0, 0)),
        out_shape=jax.ShapeDtypeStruct((N_GRAPHS, FEAT), _f32),
        scratch_shapes=[
            pltpu.VMEM((N_GRAPHS, FEAT), _f32),
            pltpu.VMEM((N_GRAPHS, 1), _f32),
        ],
    )(p, h, dis, b_row, batch_p, linw_p, linb_p)


def kernel(x, edge_index, batch, W1, b1, W2, b2, W3, b3, lin_W, lin_b):
    deg_call, spmm_call = _sc_kernels()

    sink = jnp.full((EPAD - EDGES,), N_NODES, jnp.int32)
    src_f = jnp.concatenate([edge_index[0], sink])
    dst_f = jnp.concatenate([edge_index[1], sink])
    dst_p = dst_f.reshape(NTILES, NCHUNK, CHUNK)
    sd_p = jnp.stack([src_f.reshape(TOTCH, CHUNK),
                      dst_f.reshape(TOTCH, CHUNK)], axis=1)
    x_p = jnp.pad(x, ((0, NPAD - N_NODES), (0, 0)))
    zero1 = jnp.zeros((RPT,), _f32)
    zero2 = jnp.zeros((RPT, FEAT), _f32)
    batch_p = jnp.concatenate(
        [batch, jnp.full((NPAD - N_NODES,), N_GRAPHS, batch.dtype)]
    ).reshape(NPAD, 1)
    linw_p = jnp.pad(lin_W, ((0, 0), (0, FEAT - lin_W.shape[1])))
    linb_p = jnp.broadcast_to(lin_b.reshape(1, 1), (1, FEAT))

    degp3 = deg_call(dst_p, zero1).reshape(2, NPAD, 1)
    h1, dis = _prep_call(degp3, x_p, W1)
    p1 = spmm_call(h1, sd_p, zero2)
    h2 = _combine_call(p1, h1, dis, b1.reshape(1, FEAT), W2)
    p2 = spmm_call(h2, sd_p, zero2)
    h3 = _combine_call(p2, h2, dis, b2.reshape(1, FEAT), W3)
    p3 = spmm_call(h3, sd_p, zero2)
    outm = _final_call(p3, h3, dis, b3.reshape(1, FEAT), batch_p, linw_p, linb_p)
    return outm[:, :1]
